# Initial kernel scaffold; baseline (speedup 1.0000x reference)
import functools

import jax
import jax.numpy as jnp
from jax import lax
from jax.experimental import pallas as pl
from jax.experimental.pallas import tpu as pltpu
from jax.experimental.pallas import tpu_sc as plsc

NC = 2
NS = 16
KE = 80

_MESH = plsc.VectorSubcoreMesh(
    core_axis_name="c", subcore_axis_name="s", num_cores=NC, num_subcores=NS)


def _deg_body(n_rows, n_chunks, dst_hbm, ones_hbm, zer_hbm, out_hbm,
              idx_v, ones_v, acc_sh):
    c = lax.axis_index("c")
    s = lax.axis_index("s")
    wid = c * NS + s
    pltpu.sync_copy(zer_hbm, acc_sh.at[pl.ds(s * n_rows, n_rows)])
    pltpu.sync_copy(ones_hbm, ones_v)
    pltpu.sync_copy(dst_hbm.at[wid], idx_v)
    plsc.subcore_barrier()

    def chunk(j, carry):
        pltpu.sync_copy(ones_v, acc_sh.at[idx_v.at[j]], add=True)
        return carry

    lax.fori_loop(0, n_chunks, chunk, 0)
    plsc.subcore_barrier()
    pltpu.sync_copy(acc_sh.at[pl.ds(s * n_rows, n_rows)], out_hbm.at[wid])


def _make_deg_kernel(n, e):
    kd = 40
    per_tile = e // (NC * NS)
    n_chunks = per_tile // kd
    n_rows = n // NS
    return pl.kernel(
        functools.partial(_deg_body, n_rows, n_chunks),
        out_type=jax.ShapeDtypeStruct((NC * NS, n_rows, 16), jnp.float32),
        mesh=_MESH,
        scratch_types=[
            pltpu.VMEM((n_chunks, kd), jnp.int32),
            pltpu.VMEM((kd, 16), jnp.float32),
            pltpu.VMEM_SHARED((n, 16), jnp.float32),
        ],
    )


def _agg_body(n_rows, n_chunks, src_hbm, dst_hbm, tab_hbm, zer_hbm, out_hbm,
              src_v, dst_v, rows_v, acc_sh, sem):
    c = lax.axis_index("c")
    s = lax.axis_index("s")
    wid = c * NS + s
    pltpu.sync_copy(zer_hbm, acc_sh.at[pl.ds(s * n_rows, n_rows)])
    pltpu.sync_copy(src_hbm.at[wid], src_v)
    pltpu.sync_copy(dst_hbm.at[s], dst_v)
    plsc.subcore_barrier()

    def chunk(j, carry):
        pltpu.async_copy(tab_hbm.at[src_v.at[j]], rows_v, sem).wait()
        pltpu.sync_copy(rows_v, acc_sh.at[dst_v.at[j]], add=True)
        return carry

    lax.fori_loop(0, n_chunks, chunk, 0)
    plsc.subcore_barrier()
    pltpu.sync_copy(acc_sh.at[pl.ds(s * n_rows, n_rows)], out_hbm.at[wid])


def _make_agg_kernel(n, e, dh):
    per_tile = e // NS
    n_chunks = per_tile // KE
    n_rows = n // NS
    return pl.kernel(
        functools.partial(_agg_body, n_rows, n_chunks),
        out_type=jax.ShapeDtypeStruct((NC * NS, n_rows, dh), jnp.float32),
        mesh=_MESH,
        scratch_types=[
            pltpu.VMEM((n_chunks, KE), jnp.int32),
            pltpu.VMEM((n_chunks, KE), jnp.int32),
            pltpu.VMEM((KE, dh), jnp.float32),
            pltpu.VMEM_SHARED((n, dh), jnp.float32),
            pltpu.SemaphoreType.DMA,
        ],
    )


def _dinv_from(degp_ref):
    deg = degp_ref[0, :, 0:1] + degp_ref[1, :, 0:1] + 1.0
    return lax.rsqrt(deg)


def _tc1_body(x_ref, w_ref, degp_ref, h_ref, tab_ref):
    h = jnp.dot(x_ref[...], w_ref[...], preferred_element_type=jnp.float32)
    dinv = _dinv_from(degp_ref)
    h_ref[...] = h
    ht = h * dinv
    dh = ht.shape[1] // 2
    tab_ref[0, :, :] = ht[:, :dh]
    tab_ref[1, :, :] = ht[:, dh:]


def _tc2_body(agg_ref, h1_ref, degp_ref, b1_ref, w2_ref, h2_ref, tab_ref):
    dinv = _dinv_from(degp_ref)
    agg = jnp.concatenate([agg_ref[0], agg_ref[1]], axis=1)
    z = dinv * agg + (dinv * dinv) * h1_ref[...] + b1_ref[...]
    z = jnp.maximum(z, 0.0)
    h2 = jnp.dot(z, w2_ref[...], preferred_element_type=jnp.float32)
    h2_ref[...] = h2
    ht = h2 * dinv
    dh = ht.shape[1] // 2
    tab_ref[0, :, :] = ht[:, :dh]
    tab_ref[1, :, :] = ht[:, dh:]


def _tc3_body(agg_ref, h2_ref, degp_ref, b2_ref, out_ref):
    dinv = _dinv_from(degp_ref)
    agg = jnp.concatenate([agg_ref[0], agg_ref[1]], axis=1)
    out_ref[...] = dinv * agg + (dinv * dinv) * h2_ref[...] + b2_ref[...]


def kernel(x, edge_index, W1, b1, W2, b2):
    n, d = x.shape
    e = edge_index.shape[1]
    dh = d // 2
    bn = 1000
    grid = (n // bn,)

    src = edge_index[0]
    dst = edge_index[1]
    srcoff = jnp.concatenate([src, src + n]).reshape(NC * NS, -1, KE)
    dst_agg = dst.reshape(NS, -1, KE)
    dst_deg = dst.reshape(NC * NS, -1, 40)
    zer16 = jnp.zeros((n // NS, 16), jnp.float32)
    zer128 = jnp.zeros((n // NS, dh), jnp.float32)
    ones40 = jnp.ones((40, 16), jnp.float32)

    deg_kernel = _make_deg_kernel(n, e)
    agg_kernel = _make_agg_kernel(n, e, dh)

    degp = deg_kernel(dst_deg, ones40, zer16).reshape(NC, n, 16)

    tc1 = pl.pallas_call(
        _tc1_body,
        grid=grid,
        in_specs=[
            pl.BlockSpec((bn, d), lambda i: (i, 0)),
            pl.BlockSpec((d, d), lambda i: (0, 0)),
            pl.BlockSpec((NC, bn, 16), lambda i: (0, i, 0)),
        ],
        out_specs=[
            pl.BlockSpec((bn, d), lambda i: (i, 0)),
            pl.BlockSpec((NC, bn, dh), lambda i: (0, i, 0)),
        ],
        out_shape=[
            jax.ShapeDtypeStruct((n, d), jnp.float32),
            jax.ShapeDtypeStruct((NC, n, dh), jnp.float32),
        ],
    )
    h1, tab1 = tc1(x, W1, degp)

    agg1 = agg_kernel(srcoff, dst_agg, tab1.reshape(NC * n, dh),
                      zer128).reshape(NC, n, dh)

    tc2 = pl.pallas_call(
        _tc2_body,
        grid=grid,
        in_specs=[
            pl.BlockSpec((NC, bn, dh), lambda i: (0, i, 0)),
            pl.BlockSpec((bn, d), lambda i: (i, 0)),
            pl.BlockSpec((NC, bn, 16), lambda i: (0, i, 0)),
            pl.BlockSpec((1, d), lambda i: (0, 0)),
            pl.BlockSpec((d, d), lambda i: (0, 0)),
        ],
        out_specs=[
            pl.BlockSpec((bn, d), lambda i: (i, 0)),
            pl.BlockSpec((NC, bn, dh), lambda i: (0, i, 0)),
        ],
        out_shape=[
            jax.ShapeDtypeStruct((n, d), jnp.float32),
            jax.ShapeDtypeStruct((NC, n, dh), jnp.float32),
        ],
    )
    h2, tab2 = tc2(agg1, h1, degp, b1.reshape(1, d), W2)

    agg2 = agg_kernel(srcoff, dst_agg, tab2.reshape(NC * n, dh),
                      zer128).reshape(NC, n, dh)

    tc3 = pl.pallas_call(
        _tc3_body,
        grid=grid,
        in_specs=[
            pl.BlockSpec((NC, bn, dh), lambda i: (0, i, 0)),
            pl.BlockSpec((bn, d), lambda i: (i, 0)),
            pl.BlockSpec((NC, bn, 16), lambda i: (0, i, 0)),
            pl.BlockSpec((1, d), lambda i: (0, 0)),
        ],
        out_specs=pl.BlockSpec((bn, d), lambda i: (i, 0)),
        out_shape=jax.ShapeDtypeStruct((n, d), jnp.float32),
    )
    return tc3(agg2, h2, degp, b2.reshape(1, d))


# speedup vs baseline: 10.5151x; 10.5151x over previous
import functools

import jax
import jax.numpy as jnp
from jax import lax
from jax.experimental import pallas as pl
from jax.experimental.pallas import tpu as pltpu
from jax.experimental.pallas import tpu_sc as plsc

NC = 2
NS = 16
KE = 80

_MESH = plsc.VectorSubcoreMesh(
    core_axis_name="c", subcore_axis_name="s", num_cores=NC, num_subcores=NS)


def _deg_body(n_rows, n_chunks, dst_hbm, ones_hbm, zer_hbm, out_hbm,
              idx_v, ones_v, acc_sh):
    c = lax.axis_index("c")
    s = lax.axis_index("s")
    wid = c * NS + s
    pltpu.sync_copy(zer_hbm, acc_sh.at[pl.ds(s * n_rows, n_rows)])
    pltpu.sync_copy(ones_hbm, ones_v)
    pltpu.sync_copy(dst_hbm.at[wid], idx_v)
    plsc.subcore_barrier()

    def chunk(j, carry):
        pltpu.sync_copy(ones_v, acc_sh.at[idx_v.at[j]], add=True)
        return carry

    lax.fori_loop(0, n_chunks, chunk, 0)
    plsc.subcore_barrier()
    pltpu.sync_copy(acc_sh.at[pl.ds(s * n_rows, n_rows)], out_hbm.at[wid])


def _make_deg_kernel(n, e, dh):
    kd = 40
    per_tile = e // (NC * NS)
    n_chunks = per_tile // kd
    n_rows = n // NS
    return pl.kernel(
        functools.partial(_deg_body, n_rows, n_chunks),
        out_type=jax.ShapeDtypeStruct((NC * NS, n_rows, dh), jnp.float32),
        mesh=_MESH,
        scratch_types=[
            pltpu.VMEM((n_chunks, kd), jnp.int32),
            pltpu.VMEM((kd, dh), jnp.float32),
            pltpu.VMEM_SHARED((n, dh), jnp.float32),
        ],
    )


def _agg_body(n_rows, n_chunks, src_hbm, dst_hbm, tab_hbm, zer_hbm, out_hbm,
              src_v, dst_v, rows_v, acc_sh, sem):
    c = lax.axis_index("c")
    s = lax.axis_index("s")
    wid = c * NS + s
    pltpu.sync_copy(zer_hbm, acc_sh.at[pl.ds(s * n_rows, n_rows)])
    pltpu.sync_copy(src_hbm.at[wid], src_v)
    pltpu.sync_copy(dst_hbm.at[s], dst_v)
    plsc.subcore_barrier()

    def chunk(j, carry):
        pltpu.async_copy(tab_hbm.at[src_v.at[j]], rows_v, sem).wait()
        pltpu.sync_copy(rows_v, acc_sh.at[dst_v.at[j]], add=True)
        return carry

    lax.fori_loop(0, n_chunks, chunk, 0)
    plsc.subcore_barrier()
    pltpu.sync_copy(acc_sh.at[pl.ds(s * n_rows, n_rows)], out_hbm.at[wid])


def _make_agg_kernel(n, e, dh):
    per_tile = e // NS
    n_chunks = per_tile // KE
    n_rows = n // NS
    return pl.kernel(
        functools.partial(_agg_body, n_rows, n_chunks),
        out_type=jax.ShapeDtypeStruct((NC * NS, n_rows, dh), jnp.float32),
        mesh=_MESH,
        scratch_types=[
            pltpu.VMEM((n_chunks, KE), jnp.int32),
            pltpu.VMEM((n_chunks, KE), jnp.int32),
            pltpu.VMEM((KE, dh), jnp.float32),
            pltpu.VMEM_SHARED((n, dh), jnp.float32),
            pltpu.SemaphoreType.DMA,
        ],
    )


def _dinv_from(degp_ref):
    deg = degp_ref[0, :, 0:1] + degp_ref[1, :, 0:1] + 1.0
    return lax.rsqrt(deg)


def _tc1_body(x_ref, w_ref, degp_ref, h_ref, tab_ref):
    h = jnp.dot(x_ref[...], w_ref[...], preferred_element_type=jnp.float32)
    dinv = _dinv_from(degp_ref)
    h_ref[...] = h
    ht = h * dinv
    dh = ht.shape[1] // 2
    tab_ref[0, :, :] = ht[:, :dh]
    tab_ref[1, :, :] = ht[:, dh:]


def _tc2_body(agg_ref, h1_ref, degp_ref, b1_ref, w2_ref, h2_ref, tab_ref):
    dinv = _dinv_from(degp_ref)
    agg = jnp.concatenate([agg_ref[0], agg_ref[1]], axis=1)
    z = dinv * agg + (dinv * dinv) * h1_ref[...] + b1_ref[...]
    z = jnp.maximum(z, 0.0)
    h2 = jnp.dot(z, w2_ref[...], preferred_element_type=jnp.float32)
    h2_ref[...] = h2
    ht = h2 * dinv
    dh = ht.shape[1] // 2
    tab_ref[0, :, :] = ht[:, :dh]
    tab_ref[1, :, :] = ht[:, dh:]


def _tc3_body(agg_ref, h2_ref, degp_ref, b2_ref, out_ref):
    dinv = _dinv_from(degp_ref)
    agg = jnp.concatenate([agg_ref[0], agg_ref[1]], axis=1)
    out_ref[...] = dinv * agg + (dinv * dinv) * h2_ref[...] + b2_ref[...]


def kernel(x, edge_index, W1, b1, W2, b2):
    n, d = x.shape
    e = edge_index.shape[1]
    dh = d // 2
    bn = 1000
    grid = (n // bn,)

    src = edge_index[0]
    dst = edge_index[1]
    srcoff = jnp.concatenate([src, src + n]).reshape(NC * NS, -1, KE)
    dst_agg = dst.reshape(NS, -1, KE)
    dst_deg = dst.reshape(NC * NS, -1, 40)
    zer128 = jnp.zeros((n // NS, dh), jnp.float32)
    ones40 = jnp.ones((40, dh), jnp.float32)

    deg_kernel = _make_deg_kernel(n, e, dh)
    agg_kernel = _make_agg_kernel(n, e, dh)

    degp = deg_kernel(dst_deg, ones40, zer128).reshape(NC, n, dh)

    tc1 = pl.pallas_call(
        _tc1_body,
        grid=grid,
        in_specs=[
            pl.BlockSpec((bn, d), lambda i: (i, 0)),
            pl.BlockSpec((d, d), lambda i: (0, 0)),
            pl.BlockSpec((NC, bn, dh), lambda i: (0, i, 0)),
        ],
        out_specs=[
            pl.BlockSpec((bn, d), lambda i: (i, 0)),
            pl.BlockSpec((NC, bn, dh), lambda i: (0, i, 0)),
        ],
        out_shape=[
            jax.ShapeDtypeStruct((n, d), jnp.float32),
            jax.ShapeDtypeStruct((NC, n, dh), jnp.float32),
        ],
    )
    h1, tab1 = tc1(x, W1, degp)

    agg1 = agg_kernel(srcoff, dst_agg, tab1.reshape(NC * n, dh),
                      zer128).reshape(NC, n, dh)

    tc2 = pl.pallas_call(
        _tc2_body,
        grid=grid,
        in_specs=[
            pl.BlockSpec((NC, bn, dh), lambda i: (0, i, 0)),
            pl.BlockSpec((bn, d), lambda i: (i, 0)),
            pl.BlockSpec((NC, bn, dh), lambda i: (0, i, 0)),
            pl.BlockSpec((1, d), lambda i: (0, 0)),
            pl.BlockSpec((d, d), lambda i: (0, 0)),
        ],
        out_specs=[
            pl.BlockSpec((bn, d), lambda i: (i, 0)),
            pl.BlockSpec((NC, bn, dh), lambda i: (0, i, 0)),
        ],
        out_shape=[
            jax.ShapeDtypeStruct((n, d), jnp.float32),
            jax.ShapeDtypeStruct((NC, n, dh), jnp.float32),
        ],
    )
    h2, tab2 = tc2(agg1, h1, degp, b1.reshape(1, d), W2)

    agg2 = agg_kernel(srcoff, dst_agg, tab2.reshape(NC * n, dh),
                      zer128).reshape(NC, n, dh)

    tc3 = pl.pallas_call(
        _tc3_body,
        grid=grid,
        in_specs=[
            pl.BlockSpec((NC, bn, dh), lambda i: (0, i, 0)),
            pl.BlockSpec((bn, d), lambda i: (i, 0)),
            pl.BlockSpec((NC, bn, dh), lambda i: (0, i, 0)),
            pl.BlockSpec((1, d), lambda i: (0, 0)),
        ],
        out_specs=pl.BlockSpec((bn, d), lambda i: (i, 0)),
        out_shape=jax.ShapeDtypeStruct((n, d), jnp.float32),
    )
    return tc3(agg2, h2, degp, b2.reshape(1, d))


# speedup vs baseline: 17.5217x; 1.6663x over previous
import functools

import jax
import jax.numpy as jnp
from jax import lax
from jax.experimental import pallas as pl
from jax.experimental.pallas import tpu as pltpu
from jax.experimental.pallas import tpu_sc as plsc

NC = 2
NS = 16
KE = 80

_MESH = plsc.VectorSubcoreMesh(
    core_axis_name="c", subcore_axis_name="s", num_cores=NC, num_subcores=NS)


def _deg_body(n_rows, n_chunks, dst_hbm, ones_hbm, zer_hbm, out_hbm,
              idx_v, ones_v, acc_sh):
    c = lax.axis_index("c")
    s = lax.axis_index("s")
    wid = c * NS + s
    pltpu.sync_copy(zer_hbm, acc_sh.at[pl.ds(s * n_rows, n_rows)])
    pltpu.sync_copy(ones_hbm, ones_v)
    pltpu.sync_copy(dst_hbm.at[wid], idx_v)
    plsc.subcore_barrier()

    def chunk(j, carry):
        pltpu.sync_copy(ones_v, acc_sh.at[idx_v.at[j]], add=True)
        return carry

    lax.fori_loop(0, n_chunks, chunk, 0)
    plsc.subcore_barrier()
    pltpu.sync_copy(acc_sh.at[pl.ds(s * n_rows, n_rows)], out_hbm.at[wid])


def _make_deg_kernel(n, e, dh):
    kd = 40
    per_tile = e // (NC * NS)
    n_chunks = per_tile // kd
    n_rows = n // NS
    return pl.kernel(
        functools.partial(_deg_body, n_rows, n_chunks),
        out_type=jax.ShapeDtypeStruct((NC * NS, n_rows, dh), jnp.float32),
        mesh=_MESH,
        scratch_types=[
            pltpu.VMEM((n_chunks, kd), jnp.int32),
            pltpu.VMEM((kd, dh), jnp.float32),
            pltpu.VMEM_SHARED((n, dh), jnp.float32),
        ],
        compiler_params=pltpu.CompilerParams(use_tc_tiling_on_sc=False),
    )


def _agg_body(n_rows, n_chunks, src_hbm, dst_hbm, tab_hbm, zer_hbm, out_hbm,
              src_v, dst_v, rows_a, rows_b, acc_sh, sem_a, sem_b):
    c = lax.axis_index("c")
    s = lax.axis_index("s")
    wid = c * NS + s
    pltpu.sync_copy(zer_hbm, acc_sh.at[pl.ds(s * n_rows, n_rows)])
    pltpu.sync_copy(src_hbm.at[wid], src_v)
    pltpu.sync_copy(dst_hbm.at[s], dst_v)
    plsc.subcore_barrier()

    pltpu.async_copy(tab_hbm.at[src_v.at[0]], rows_a, sem_a)

    def pair(i, carry):
        j = i * 2
        pltpu.async_copy(tab_hbm.at[src_v.at[j + 1]], rows_b, sem_b)
        pltpu.make_async_copy(tab_hbm.at[src_v.at[j]], rows_a, sem_a).wait()
        pltpu.sync_copy(rows_a, acc_sh.at[dst_v.at[j]], add=True)
        pltpu.async_copy(tab_hbm.at[src_v.at[j + 2]], rows_a, sem_a)
        pltpu.make_async_copy(tab_hbm.at[src_v.at[j + 1]], rows_b,
                              sem_b).wait()
        pltpu.sync_copy(rows_b, acc_sh.at[dst_v.at[j + 1]], add=True)
        return carry

    lax.fori_loop(0, n_chunks // 2, pair, 0)
    last = n_chunks - 1
    pltpu.make_async_copy(tab_hbm.at[src_v.at[last]], rows_a, sem_a).wait()
    pltpu.sync_copy(rows_a, acc_sh.at[dst_v.at[last]], add=True)
    plsc.subcore_barrier()
    pltpu.sync_copy(acc_sh.at[pl.ds(s * n_rows, n_rows)], out_hbm.at[wid])


def _make_agg_kernel(n, e, dh):
    per_tile = e // NS
    n_chunks = per_tile // KE
    n_rows = n // NS
    return pl.kernel(
        functools.partial(_agg_body, n_rows, n_chunks),
        out_type=jax.ShapeDtypeStruct((NC * NS, n_rows, dh), jnp.float32),
        mesh=_MESH,
        scratch_types=[
            pltpu.VMEM((n_chunks, KE), jnp.int32),
            pltpu.VMEM((n_chunks, KE), jnp.int32),
            pltpu.VMEM((KE, dh), jnp.float32),
            pltpu.VMEM((KE, dh), jnp.float32),
            pltpu.VMEM_SHARED((n, dh), jnp.float32),
            pltpu.SemaphoreType.DMA,
            pltpu.SemaphoreType.DMA,
        ],
        compiler_params=pltpu.CompilerParams(use_tc_tiling_on_sc=False),
    )


def _dinv_from(degp_ref):
    deg = degp_ref[0, :, 0:1] + degp_ref[1, :, 0:1] + 1.0
    return lax.rsqrt(deg)


def _tc1_body(x_ref, w_ref, degp_ref, h_ref, tab_ref):
    h = jnp.dot(x_ref[...], w_ref[...], preferred_element_type=jnp.float32)
    dinv = _dinv_from(degp_ref)
    h_ref[...] = h
    ht = h * dinv
    dh = ht.shape[1] // 2
    tab_ref[0, :, :] = ht[:, :dh]
    tab_ref[1, :, :] = ht[:, dh:]


def _tc2_body(agg_ref, h1_ref, degp_ref, b1_ref, w2_ref, h2_ref, tab_ref):
    dinv = _dinv_from(degp_ref)
    agg = jnp.concatenate([agg_ref[0], agg_ref[1]], axis=1)
    z = dinv * agg + (dinv * dinv) * h1_ref[...] + b1_ref[...]
    z = jnp.maximum(z, 0.0)
    h2 = jnp.dot(z, w2_ref[...], preferred_element_type=jnp.float32)
    h2_ref[...] = h2
    ht = h2 * dinv
    dh = ht.shape[1] // 2
    tab_ref[0, :, :] = ht[:, :dh]
    tab_ref[1, :, :] = ht[:, dh:]


def _tc3_body(agg_ref, h2_ref, degp_ref, b2_ref, out_ref):
    dinv = _dinv_from(degp_ref)
    agg = jnp.concatenate([agg_ref[0], agg_ref[1]], axis=1)
    out_ref[...] = dinv * agg + (dinv * dinv) * h2_ref[...] + b2_ref[...]


def kernel(x, edge_index, W1, b1, W2, b2):
    n, d = x.shape
    e = edge_index.shape[1]
    dh = d // 2
    bn = 1000
    grid = (n // bn,)

    src = edge_index[0]
    dst = edge_index[1]
    srcoff = jnp.concatenate([src, src + n]).reshape(NC * NS, -1, KE)
    dst_agg = dst.reshape(NS, -1, KE)
    dst_deg = dst.reshape(NC * NS, -1, 40)
    zer128 = jnp.zeros((n // NS, dh), jnp.float32)
    zer16 = jnp.zeros((n // NS, 16), jnp.float32)
    ones40 = jnp.ones((40, 16), jnp.float32)

    deg_kernel = _make_deg_kernel(n, e, 16)
    agg_kernel = _make_agg_kernel(n, e, dh)

    degp = deg_kernel(dst_deg, ones40, zer16).reshape(NC, n, 16)

    tc1 = pl.pallas_call(
        _tc1_body,
        grid=grid,
        in_specs=[
            pl.BlockSpec((bn, d), lambda i: (i, 0)),
            pl.BlockSpec((d, d), lambda i: (0, 0)),
            pl.BlockSpec((NC, bn, 16), lambda i: (0, i, 0)),
        ],
        out_specs=[
            pl.BlockSpec((bn, d), lambda i: (i, 0)),
            pl.BlockSpec((NC, bn, dh), lambda i: (0, i, 0)),
        ],
        out_shape=[
            jax.ShapeDtypeStruct((n, d), jnp.float32),
            jax.ShapeDtypeStruct((NC, n, dh), jnp.float32),
        ],
    )
    h1, tab1 = tc1(x, W1, degp)

    agg1 = agg_kernel(srcoff, dst_agg, tab1.reshape(NC * n, dh),
                      zer128).reshape(NC, n, dh)

    tc2 = pl.pallas_call(
        _tc2_body,
        grid=grid,
        in_specs=[
            pl.BlockSpec((NC, bn, dh), lambda i: (0, i, 0)),
            pl.BlockSpec((bn, d), lambda i: (i, 0)),
            pl.BlockSpec((NC, bn, 16), lambda i: (0, i, 0)),
            pl.BlockSpec((1, d), lambda i: (0, 0)),
            pl.BlockSpec((d, d), lambda i: (0, 0)),
        ],
        out_specs=[
            pl.BlockSpec((bn, d), lambda i: (i, 0)),
            pl.BlockSpec((NC, bn, dh), lambda i: (0, i, 0)),
        ],
        out_shape=[
            jax.ShapeDtypeStruct((n, d), jnp.float32),
            jax.ShapeDtypeStruct((NC, n, dh), jnp.float32),
        ],
    )
    h2, tab2 = tc2(agg1, h1, degp, b1.reshape(1, d), W2)

    agg2 = agg_kernel(srcoff, dst_agg, tab2.reshape(NC * n, dh),
                      zer128).reshape(NC, n, dh)

    tc3 = pl.pallas_call(
        _tc3_body,
        grid=grid,
        in_specs=[
            pl.BlockSpec((NC, bn, dh), lambda i: (0, i, 0)),
            pl.BlockSpec((bn, d), lambda i: (i, 0)),
            pl.BlockSpec((NC, bn, 16), lambda i: (0, i, 0)),
            pl.BlockSpec((1, d), lambda i: (0, 0)),
        ],
        out_specs=pl.BlockSpec((bn, d), lambda i: (i, 0)),
        out_shape=jax.ShapeDtypeStruct((n, d), jnp.float32),
    )
    return tc3(agg2, h2, degp, b2.reshape(1, d))
